# baseline (speedup 1.0000x reference)
import functools

import jax
import jax.numpy as jnp
from jax import lax
from jax.experimental import pallas as pl
from jax.experimental.pallas import tpu as pltpu
from jax.experimental.pallas import tpu_sc as plsc

DIM0 = 1024
N_CLS = 1000
N_CTX = 16
SUF_LEN = 60
CTX_DIM = 768
W32 = CTX_DIM // 2
ROW = 1 + N_CTX + SUF_LEN

NC = 2
NS = 16
NW = NC * NS
BPW = DIM0 // NW

_mesh = plsc.VectorSubcoreMesh(core_axis_name="c", subcore_axis_name="s")


@functools.partial(
    pl.kernel,
    mesh=_mesh,
    out_type=jax.ShapeDtypeStruct((DIM0, ROW, W32), jnp.int32),
    compiler_params=pltpu.CompilerParams(use_tc_tiling_on_sc=False,
                                         needs_layout_passes=False),
    scratch_types=[
        pltpu.VMEM((DIM0,), jnp.int32),
        pltpu.VMEM((BPW, W32), jnp.int32),
        pltpu.VMEM((81, W32), jnp.int32),
        pltpu.VMEM((81, W32), jnp.int32),
        pltpu.SemaphoreType.DMA,
        pltpu.SemaphoreType.DMA,
        pltpu.SemaphoreType.DMA,
        pltpu.SemaphoreType.DMA,
        pltpu.SemaphoreType.DMA,
    ],
)
def _sc_assemble(pre_hbm, suf_hbm, ctx_hbm, lbl_hbm, out_hbm,
                 lblv, prebuf, buf0, buf1, semL0, semL1, semW0, semW1, semP):
    wid = lax.axis_index("s") * NC + lax.axis_index("c")
    base = wid * BPW

    pltpu.sync_copy(lbl_hbm, lblv)

    pltpu.async_copy(pre_hbm.at[lblv.at[pl.ds(base, BPW)]], prebuf,
                     semP).wait()

    bufs = (buf0, buf1)
    semL = (semL0, semL1)
    semW = (semW0, semW1)
    iota = lax.iota(jnp.int32, 16)

    def issue_loads(i):
        s = i % 2
        b = base + i
        lbl_b = plsc.load_gather(lblv, [jnp.full((16,), base + i, jnp.int32)])
        ds_ = [pltpu.async_copy(ctx_hbm.at[b], bufs[s].at[pl.ds(1, N_CTX)],
                                semL[s])]
        for v in range(4):
            tok = jnp.minimum(iota + (v * 16), SUF_LEN - 1)
            idx = lbl_b * SUF_LEN + tok
            ds_.append(pltpu.async_copy(
                suf_hbm.at[idx], bufs[s].at[pl.ds(1 + N_CTX + v * 16, 16)],
                semL[s]))
        return ds_

    descs = [None, None]
    wdesc = [None, None]
    descs[0] = issue_loads(0)
    for i in range(BPW):
        s = i % 2
        if i + 1 < BPW:
            s2 = 1 - s
            if wdesc[s2] is not None:
                wdesc[s2].wait()
            descs[s2] = issue_loads(i + 1)
        for d in descs[s]:
            d.wait()
        for v in range(W32 // 16):
            bufs[s][0, pl.ds(v * 16, 16)] = prebuf[i, pl.ds(v * 16, 16)]
        wdesc[s] = pltpu.async_copy(
            bufs[s].at[pl.ds(0, ROW)], out_hbm.at[base + i], semW[s])
    wdesc[0].wait()
    wdesc[1].wait()


def kernel(ctx, prefix, suffix, label):
    ctx16 = ctx.astype(jnp.float16)
    ctx_i = jax.lax.bitcast_convert_type(
        ctx16.reshape(DIM0, N_CTX, W32, 2), jnp.int32)
    pre_i = jax.lax.bitcast_convert_type(
        prefix.reshape(N_CLS, W32, 2), jnp.int32)
    suf_i = jax.lax.bitcast_convert_type(
        suffix.reshape(N_CLS * SUF_LEN, W32, 2), jnp.int32)
    lbl = label.astype(jnp.int32)
    out_i = _sc_assemble(pre_i, suf_i, ctx_i, lbl)
    out = jax.lax.bitcast_convert_type(out_i, jnp.float16)
    return out.reshape(DIM0, ROW, CTX_DIM)
